# baseline (speedup 1.0000x reference)
import functools

import jax
import jax.numpy as jnp
from jax import lax
from jax.experimental import pallas as pl
from jax.experimental.pallas import tpu as pltpu
from jax.experimental.pallas import tpu_sc as plsc

N = 10000
E = 320000
F = 128
DE = 16
H = 128
KSEL = N // 2

NP = 10240
EP = 323584
CB = 64
TE = EP // 32
NR = (NP // 16) // CB
DUMMY = N
KP = 5120


def _mm_body(a_ref, w_ref, o_ref):
    o_ref[...] = jnp.dot(a_ref[...], w_ref[...],
                         preferred_element_type=jnp.float32)


def _mm(a, w, rb):
    m, k = a.shape
    _, h = w.shape
    return pl.pallas_call(
        _mm_body,
        grid=(m // rb,),
        in_specs=[pl.BlockSpec((rb, k), lambda i: (i, 0)),
                  pl.BlockSpec((k, h), lambda i: (0, 0))],
        out_specs=pl.BlockSpec((rb, h), lambda i: (i, 0)),
        out_shape=jax.ShapeDtypeStruct((m, h), jnp.float32),
    )(a, w)


def _mm_bias_body(a_ref, w_ref, b_ref, o_ref):
    o_ref[...] = jnp.dot(a_ref[...], w_ref[...],
                         preferred_element_type=jnp.float32) + b_ref[...]


def _mm_bias(a, w, b, rb):
    m, k = a.shape
    _, h = w.shape
    return pl.pallas_call(
        _mm_bias_body,
        grid=(m // rb,),
        in_specs=[pl.BlockSpec((rb, k), lambda i: (i, 0)),
                  pl.BlockSpec((k, h), lambda i: (0, 0)),
                  pl.BlockSpec((1, h), lambda i: (0, 0))],
        out_specs=pl.BlockSpec((rb, h), lambda i: (i, 0)),
        out_shape=jax.ShapeDtypeStruct((m, h), jnp.float32),
    )(a, w, b.reshape(1, h))


_RB = 1024


def _nu_score_body(a_ref, w_ref, g0_ref, pw_ref, x_ref, s_ref):
    acc = jnp.dot(a_ref[...], w_ref[...], preferred_element_type=jnp.float32)
    x1 = jnp.maximum(acc + g0_ref[...], 0.0)
    x_ref[...] = x1
    s = jnp.dot(x1, pw_ref[...], preferred_element_type=jnp.float32)
    rows = pl.program_id(0) * _RB + lax.broadcasted_iota(jnp.int32, (_RB, 1), 0)
    s_ref[...] = jnp.where(rows < N, s, -1e30)


def _node_update_score(a, w, g0, pw):
    m, k = a.shape
    _, h = w.shape
    return pl.pallas_call(
        _nu_score_body,
        grid=(m // _RB,),
        in_specs=[pl.BlockSpec((_RB, k), lambda i: (i, 0)),
                  pl.BlockSpec((k, h), lambda i: (0, 0)),
                  pl.BlockSpec((_RB, h), lambda i: (i, 0)),
                  pl.BlockSpec((h, 1), lambda i: (0, 0))],
        out_specs=[pl.BlockSpec((_RB, h), lambda i: (i, 0)),
                   pl.BlockSpec((_RB, 1), lambda i: (i, 0))],
        out_shape=[jax.ShapeDtypeStruct((m, h), jnp.float32),
                   jax.ShapeDtypeStruct((m, 1), jnp.float32)],
    )(a, w, g0, pw.reshape(h, 1))


def _nu2_body(a_ref, w_ref, r0_ref, r1_ref, r2_ref, r3_ref, o_ref):
    acc = jnp.dot(a_ref[...], w_ref[...], preferred_element_type=jnp.float32)
    agg = jnp.concatenate([r0_ref[...] + r1_ref[...],
                           r2_ref[...] + r3_ref[...]], axis=1)
    o_ref[...] = jnp.maximum(acc + agg, 0.0)


def _node_update2(a, w, r0, r1, r2, r3):
    m, k = a.shape
    _, h = w.shape
    hh = h // 2
    return pl.pallas_call(
        _nu2_body,
        grid=(m // _RB,),
        in_specs=[pl.BlockSpec((_RB, k), lambda i: (i, 0)),
                  pl.BlockSpec((k, h), lambda i: (0, 0)),
                  pl.BlockSpec((_RB, hh), lambda i: (i, 0)),
                  pl.BlockSpec((_RB, hh), lambda i: (i, 0)),
                  pl.BlockSpec((_RB, hh), lambda i: (i, 0)),
                  pl.BlockSpec((_RB, hh), lambda i: (i, 0))],
        out_specs=pl.BlockSpec((_RB, h), lambda i: (i, 0)),
        out_shape=jax.ShapeDtypeStruct((m, h), jnp.float32),
    )(a, w, r0, r1, r2, r3)


def _nu3_body(a_ref, w_ref, r0_ref, r1_ref, o_ref):
    acc = jnp.dot(a_ref[...], w_ref[...], preferred_element_type=jnp.float32)
    o_ref[...] = jnp.maximum(acc + r0_ref[...] + r1_ref[...], 0.0)


def _node_update3(a, w, r0, r1):
    m, k = a.shape
    _, h = w.shape
    return pl.pallas_call(
        _nu3_body,
        grid=(m // _RB,),
        in_specs=[pl.BlockSpec((_RB, k), lambda i: (i, 0)),
                  pl.BlockSpec((k, h), lambda i: (0, 0)),
                  pl.BlockSpec((_RB, h), lambda i: (i, 0)),
                  pl.BlockSpec((_RB, h), lambda i: (i, 0))],
        out_specs=pl.BlockSpec((_RB, h), lambda i: (i, 0)),
        out_shape=jax.ShapeDtypeStruct((m, h), jnp.float32),
    )(a, w, r0, r1)


_JB = 1024


def _rank_body(scol_ref, srow_ref, x1_ref, pw_ref, xg_ref, kf_ref, rank_ref):
    si = scol_ref[...]
    ii = pl.program_id(0) * _RB + lax.broadcasted_iota(jnp.int32, (_RB, 1), 0)

    def body(jc, acc):
        sj = srow_ref[:, pl.ds(jc * _JB, _JB)]
        jj = jc * _JB + lax.broadcasted_iota(jnp.int32, (1, _JB), 1)
        gt = (sj > si).astype(jnp.int32)
        eq = jnp.logical_and(sj == si, jj < ii).astype(jnp.int32)
        return acc + jnp.sum(gt + eq, axis=1, keepdims=True)

    rank = lax.fori_loop(0, NP // _JB, body, jnp.zeros((_RB, 1), jnp.int32))
    rank_ref[...] = rank
    kf_ref[...] = (rank < KSEL).astype(jnp.int32)
    wn = jnp.sqrt(jnp.sum(pw_ref[...] ** 2))
    gate = jnp.tanh(si / (wn + 1e-16))
    xg_ref[...] = x1_ref[...] * gate


def _rank_gate(score, x1, pool_w):
    h = x1.shape[1]
    return pl.pallas_call(
        _rank_body,
        grid=(NP // _RB,),
        in_specs=[pl.BlockSpec((_RB, 1), lambda i: (i, 0)),
                  pl.BlockSpec((1, NP), lambda i: (0, 0)),
                  pl.BlockSpec((_RB, h), lambda i: (i, 0)),
                  pl.BlockSpec((1, h), lambda i: (0, 0))],
        out_specs=[pl.BlockSpec((_RB, h), lambda i: (i, 0)),
                   pl.BlockSpec((_RB, 1), lambda i: (i, 0)),
                   pl.BlockSpec((_RB, 1), lambda i: (i, 0))],
        out_shape=[jax.ShapeDtypeStruct((NP, h), jnp.float32),
                   jax.ShapeDtypeStruct((NP, 1), jnp.int32),
                   jax.ShapeDtypeStruct((NP, 1), jnp.int32)],
    )(score, score.reshape(1, NP), x1, pool_w.reshape(1, h))


def _mk_splits():
    bs = []
    for half in (0, 160000):
        for kk in range(1, 12):
            bs.append(half + kk * 10080)
        for mm in range(1, 5):
            bs.append(half + 110880 + mm * 9840)
    bs.append(160000)
    return tuple(sorted(bs))


_B_SPLITS = _mk_splits()
_NT = NP // 32
_ELCAP = 12288
_SCN = 512
_EB = 64


def _edge_pass_exact(table, ea, src, dst, b):
    mesh = plsc.VectorSubcoreMesh(core_axis_name="c", subcore_axis_name="s")

    @functools.partial(
        pl.kernel,
        out_type=jax.ShapeDtypeStruct((NP, H), jnp.float32),
        mesh=mesh,
        compiler_params=pltpu.CompilerParams(needs_layout_passes=False),
        scratch_types=[
            pltpu.VMEM((_NT, H), jnp.float32),
            pltpu.VMEM((_NT, H), jnp.float32),
            pltpu.VMEM((_ELCAP,), jnp.int32),
            pltpu.VMEM((_SCN,), jnp.int32),
            pltpu.VMEM((_EB, H), jnp.float32),
            pltpu.VMEM((_EB, H), jnp.float32),
            pltpu.VMEM((_EB, H), jnp.float32),
            pltpu.VMEM((_EB, H), jnp.float32),
            pltpu.VMEM((_EB,), jnp.int32),
            pltpu.VMEM((_EB,), jnp.int32),
            pltpu.VMEM((_EB,), jnp.int32),
            pltpu.VMEM((_EB,), jnp.int32),
            pltpu.VMEM((_NT,), jnp.int32),
            pltpu.SMEM((4,), jnp.int32),
            pltpu.SMEM((4,), jnp.int32),
            pltpu.SMEM((4,), jnp.int32),
            pltpu.VMEM((H,), jnp.float32),
            pltpu.SemaphoreType.DMA,
            pltpu.SemaphoreType.DMA,
            pltpu.SemaphoreType.DMA,
        ],
    )
    def k(table_h, ea_h, src_h, dst_h, b_h, out_h,
          acc, part1, elist, dch, rows_a, rows_b, ea_a, ea_b,
          sidx_a, sidx_b, didx_a, didx_b,
          hist, fl_node, fl_tgt, fl_seen, b_v, s1sem, rs_a, rs_b):
        cid = lax.axis_index("c")
        sid = lax.axis_index("s")
        wid = cid * 16 + sid
        lo = wid * _NT

        pltpu.sync_copy(b_h, b_v)

        def z2(i, _):
            for j in range(H // 16):
                acc[i, pl.ds(j * 16, 16)] = jnp.zeros((16,), jnp.float32)
                part1[i, pl.ds(j * 16, 16)] = jnp.zeros((16,), jnp.float32)
            return 0
        lax.fori_loop(0, _NT, z2, 0)
        for t in range(_NT // 16):
            hist[pl.ds(t * 16, 16)] = jnp.zeros((16,), jnp.int32)

        def zel(i, _):
            elist[pl.ds(i * 16, 16)] = jnp.zeros((16,), jnp.int32)
            return 0
        lax.fori_loop(0, _ELCAP // 16, zel, 0)
        for si in range(3):
            fl_node[si] = -1
            fl_tgt[si] = -1
            fl_seen[si] = 0

        def scan_chunk(c, carry):
            pltpu.sync_copy(dst_h.at[pl.ds(c * _SCN, _SCN)], dch)

            def grp(g, carry2):
                wp2, cb2 = carry2
                dv = dch[pl.ds(g * 16, 16)]
                in_rng = jnp.logical_and(dv >= lo, dv < lo + _NT)
                in_rng = jnp.logical_and(in_rng, dv < N)
                plsc.addupdate_scatter(hist, [dv - lo],
                                       jnp.ones((16,), jnp.int32),
                                       mask=in_rng)
                below = plsc.all_reduce_population_count(dv < lo)[0]
                eids = (c * _SCN + g * 16
                        + lax.broadcasted_iota(jnp.int32, (16,), 0))
                plsc.store_compressed(elist.at[pl.ds(wp2, 16)], eids,
                                      mask=in_rng)
                nin = plsc.all_reduce_population_count(in_rng)[0]
                return (wp2 + nin, cb2 + below)
            return lax.fori_loop(0, _SCN // 16, grp, carry)
        wp, cbelow = lax.fori_loop(0, E // _SCN, scan_chunk,
                                   (jnp.int32(0), jnp.int32(0)))

        def ph2(t, carry):
            pos, kslot = carry
            h16 = hist[pl.ds(t * 16, 16)]
            inc = plsc.cumsum(h16)
            st = pos + (inc - h16)
            en = pos + inc
            csp = jnp.full((16,), -1, jnp.int32)
            for bt in _B_SPLITS:
                cond = jnp.logical_and(st < bt, bt < en)
                csp = jnp.where(cond, bt - st, csp)
            m = csp >= 0
            kcnt = plsc.all_reduce_population_count(m)[0]
            idx16 = t * 16 + lax.broadcasted_iota(jnp.int32, (16,), 0)
            nid = jnp.sum(jnp.where(m, idx16, 0))
            cc = jnp.sum(jnp.where(m, csp, 0))

            @pl.when(kcnt > 0)
            def _():
                fl_node[kslot] = nid
                fl_tgt[kslot] = cc
            return (pos + jnp.sum(h16), jnp.minimum(kslot + kcnt, 2))
        _, _ = lax.fori_loop(0, _NT // 16, ph2, (cbelow, jnp.int32(0)))
        f0n = fl_node[0]
        f1n = fl_node[1]
        f2n = fl_node[2]

        nblocks = (wp + _EB - 1) // _EB

        def issue_s1(base, sidx_x, didx_x, ea_x):
            il = elist.at[pl.ds(base, _EB)]
            d1 = pltpu.async_copy(src_h.at[il], sidx_x, s1sem)
            d2 = pltpu.async_copy(dst_h.at[il], didx_x, s1sem)
            d3 = pltpu.async_copy(ea_h.at[il], ea_x, s1sem)
            d1.wait()
            d2.wait()
            d3.wait()

        def issue_rows(sidx_x, rows_x, rs_x):
            pltpu.async_copy(table_h.at[sidx_x], rows_x, rs_x)

        def wait_rows(sidx_x, rows_x, rs_x):
            pltpu.make_async_copy(table_h.at[sidx_x], rows_x, rs_x).wait()

        def pedge_f(base, rows_x, ea_x, didx_x):
            def accum(ld, i):
                for jj in range(H // 16):
                    m = ((rows_x[i, pl.ds(jj * 16, 16)]
                          + ea_x[i, pl.ds(jj * 16, 16)])
                         + b_v[pl.ds(jj * 16, 16)])
                    acc[ld, pl.ds(jj * 16, 16)] = (
                        acc[ld, pl.ds(jj * 16, 16)]
                        + jnp.maximum(m, 0.0))

            def pedge_grp(g, _):
                gbase = base + g * 16
                dv16 = didx_x[pl.ds(g * 16, 16)] - lo
                hit16 = jnp.logical_or(
                    dv16 == f0n, jnp.logical_or(dv16 == f1n, dv16 == f2n))
                hit16i = hit16.astype(jnp.int32)
                nhit = plsc.all_reduce_population_count(hit16)[0]
                plain = jnp.logical_and(gbase + 16 <= wp, nhit == 0)

                @pl.when(plain)
                def _():
                    for j in range(16):
                        accum(dv16[j], g * 16 + j)

                @pl.when(jnp.logical_not(plain))
                def _():
                    for j in range(16):
                        i = g * 16 + j

                        @pl.when(gbase + j < wp)
                        def _(i=i, j=j):
                            ld = dv16[j]

                            @pl.when(hit16i[j] == 1)
                            def _():
                                slot = jnp.where(
                                    ld == f0n, 0,
                                    jnp.where(ld == f1n, 1, 2))
                                s = fl_seen[slot]

                                @pl.when(s == fl_tgt[slot])
                                def _():
                                    for jj in range(H // 16):
                                        part1[ld, pl.ds(jj * 16, 16)] = (
                                            part1[ld, pl.ds(jj * 16, 16)]
                                            + acc[ld, pl.ds(jj * 16, 16)])
                                        acc[ld, pl.ds(jj * 16, 16)] = (
                                            jnp.zeros((16,), jnp.float32))
                                fl_seen[slot] = s + 1
                            accum(ld, i)
                return 0
            lax.fori_loop(0, _EB // 16, pedge_grp, 0)

        issue_s1(0, sidx_a, didx_a, ea_a)
        issue_rows(sidx_a, rows_a, rs_a)

        def it(ii, _):
            b0 = 2 * ii
            b1 = b0 + 1
            wait_rows(sidx_a, rows_a, rs_a)

            @pl.when(b1 < nblocks)
            def _():
                issue_s1(b1 * _EB, sidx_b, didx_b, ea_b)
                issue_rows(sidx_b, rows_b, rs_b)
            pedge_f(b0 * _EB, rows_a, ea_a, didx_a)

            @pl.when(b1 < nblocks)
            def _():
                wait_rows(sidx_b, rows_b, rs_b)

                @pl.when(b1 + 1 < nblocks)
                def _():
                    issue_s1((b1 + 1) * _EB, sidx_a, didx_a, ea_a)
                    issue_rows(sidx_a, rows_a, rs_a)
                pedge_f(b1 * _EB, rows_b, ea_b, didx_b)
            return 0
        lax.fori_loop(0, (nblocks + 1) // 2, it, 0)

        def ep(t, _):
            def row(r, _):
                for j in range(H // 16):
                    rows_a[r, pl.ds(j * 16, 16)] = (
                        part1[t * _EB + r, pl.ds(j * 16, 16)]
                        + acc[t * _EB + r, pl.ds(j * 16, 16)])
                return 0
            lax.fori_loop(0, _EB, row, 0)
            pltpu.sync_copy(rows_a, out_h.at[pl.ds(lo + t * _EB, _EB)])
            return 0
        lax.fori_loop(0, _NT // _EB, ep, 0)

    return k(table, ea, src, dst, b)


def _edge_pass(table, ea, src, dst, kf, hh):
    mesh = plsc.VectorSubcoreMesh(core_axis_name="c", subcore_axis_name="s")

    @functools.partial(
        pl.kernel,
        out_type=jax.ShapeDtypeStruct((2 * NP, hh), jnp.float32),
        mesh=mesh,
        compiler_params=pltpu.CompilerParams(needs_layout_passes=False),
        scratch_types=[
            pltpu.VMEM((NP,), jnp.int32),
            pltpu.VMEM((CB,), jnp.int32),
            pltpu.VMEM((CB,), jnp.int32),
            pltpu.VMEM((CB,), jnp.int32),
            pltpu.VMEM((CB,), jnp.int32),
            pltpu.VMEM((CB,), jnp.int32),
            pltpu.VMEM((CB,), jnp.int32),
            pltpu.VMEM((CB, hh), jnp.float32),
            pltpu.VMEM((CB, hh), jnp.float32),
            pltpu.VMEM((CB, hh), jnp.float32),
            pltpu.VMEM((CB, hh), jnp.float32),
            pltpu.VMEM_SHARED((NP, hh), jnp.float32),
            pltpu.SemaphoreType.DMA,
            pltpu.SemaphoreType.DMA,
            pltpu.SemaphoreType.DMA,
            pltpu.SemaphoreType.DMA,
        ],
    )
    def k(table_h, ea_h, src_h, dst_h, kf_h, out_h,
          kf_v, src_a, src_b, dst_a, dst_b, idx_a, idx_b,
          rows_a, rows_b, ea_a, ea_b, agg_sh,
          s1_a, s1_b, rs_a, rs_b):
        cid = lax.axis_index("c")
        sid = lax.axis_index("s")
        wid = cid * 16 + sid
        nc = TE // CB

        pltpu.sync_copy(kf_h, kf_v)

        def zrow(i, _):
            for j in range(hh // 16):
                rows_a[i, pl.ds(j * 16, 16)] = jnp.zeros((16,), jnp.float32)
            return 0
        lax.fori_loop(0, CB, zrow, 0)
        for t in range(NR):
            pltpu.sync_copy(rows_a,
                            agg_sh.at[pl.ds(sid * (NP // 16) + t * CB, CB)])
        plsc.subcore_barrier()

        def issue_s1(ci, src_x, dst_x, ea_x, s1_x):
            ebase = wid * TE + ci * CB
            pltpu.async_copy(src_h.at[pl.ds(ebase, CB)], src_x, s1_x)
            pltpu.async_copy(dst_h.at[pl.ds(ebase, CB)], dst_x, s1_x)
            pltpu.async_copy(ea_h.at[pl.ds(ebase, CB)], ea_x, s1_x)

        def wait_s1(ci, src_x, dst_x, ea_x, s1_x):
            ebase = wid * TE + ci * CB
            pltpu.make_async_copy(src_h.at[pl.ds(ebase, CB)], src_x, s1_x).wait()
            pltpu.make_async_copy(dst_h.at[pl.ds(ebase, CB)], dst_x, s1_x).wait()
            pltpu.make_async_copy(ea_h.at[pl.ds(ebase, CB)], ea_x, s1_x).wait()

        issue_s1(0, src_a, dst_a, ea_a, s1_a)

        def it(ii, _):
            c0 = 2 * ii
            c1 = c0 + 1
            @pl.when(c0 == 0)
            def _():
                wait_s1(0, src_a, dst_a, ea_a, s1_a)
                pltpu.async_copy(table_h.at[src_a], rows_a, rs_a)
                issue_s1(1, src_b, dst_b, ea_b, s1_b)
                for j in range(CB // 16):
                    sv = src_a[pl.ds(j * 16, 16)]
                    dv = dst_a[pl.ds(j * 16, 16)]
                    ks = plsc.load_gather(kf_v, [sv])
                    kd = plsc.load_gather(kf_v, [dv])
                    ok = (ks + kd) == 2
                    idx_a[pl.ds(j * 16, 16)] = jnp.where(ok, dv, DUMMY + wid)
                pltpu.make_async_copy(table_h.at[src_a], rows_a, rs_a).wait()

                def mrow0(r, _):
                    for j in range(hh // 16):
                        v = (rows_a[r, pl.ds(j * 16, 16)]
                             + ea_a[r, pl.ds(j * 16, 16)])
                        ea_a[r, pl.ds(j * 16, 16)] = jnp.maximum(v, 0.0)
                    return 0
                lax.fori_loop(0, CB, mrow0, 0)
                pltpu.sync_copy(ea_a, agg_sh.at[idx_a], add=True)

            @pl.when(c0 > 0)
            def _():
                wait_s1(c0, src_a, dst_a, ea_a, s1_a)
                pltpu.async_copy(table_h.at[src_a], rows_a, rs_a)

                @pl.when(c1 < nc)
                def _():
                    issue_s1(c1, src_b, dst_b, ea_b, s1_b)
                for j in range(CB // 16):
                    sv = src_a[pl.ds(j * 16, 16)]
                    dv = dst_a[pl.ds(j * 16, 16)]
                    ks = plsc.load_gather(kf_v, [sv])
                    kd = plsc.load_gather(kf_v, [dv])
                    ok = (ks + kd) == 2
                    idx_a[pl.ds(j * 16, 16)] = jnp.where(ok, dv, DUMMY + wid)
                pltpu.make_async_copy(table_h.at[src_a], rows_a, rs_a).wait()

                def mrow1(r, _):
                    for j in range(hh // 16):
                        v = (rows_a[r, pl.ds(j * 16, 16)]
                             + ea_a[r, pl.ds(j * 16, 16)])
                        ea_a[r, pl.ds(j * 16, 16)] = jnp.maximum(v, 0.0)
                    return 0
                lax.fori_loop(0, CB, mrow1, 0)
                pltpu.sync_copy(ea_a, agg_sh.at[idx_a], add=True)

            @pl.when(c1 < nc)
            def _():
                wait_s1(c1, src_b, dst_b, ea_b, s1_b)
                pltpu.async_copy(table_h.at[src_b], rows_b, rs_b)

                @pl.when(c1 + 1 < nc)
                def _():
                    issue_s1(c1 + 1, src_a, dst_a, ea_a, s1_a)
                for j in range(CB // 16):
                    sv = src_b[pl.ds(j * 16, 16)]
                    dv = dst_b[pl.ds(j * 16, 16)]
                    ks = plsc.load_gather(kf_v, [sv])
                    kd = plsc.load_gather(kf_v, [dv])
                    ok = (ks + kd) == 2
                    idx_b[pl.ds(j * 16, 16)] = jnp.where(ok, dv, DUMMY + wid)
                pltpu.make_async_copy(table_h.at[src_b], rows_b, rs_b).wait()

                def mrow2(r, _):
                    for j in range(hh // 16):
                        v = (rows_b[r, pl.ds(j * 16, 16)]
                             + ea_b[r, pl.ds(j * 16, 16)])
                        ea_b[r, pl.ds(j * 16, 16)] = jnp.maximum(v, 0.0)
                    return 0
                lax.fori_loop(0, CB, mrow2, 0)
                pltpu.sync_copy(ea_b, agg_sh.at[idx_b], add=True)
            return 0
        lax.fori_loop(0, (nc + 1) // 2, it, 0)

        plsc.subcore_barrier()
        for t in range(NR):
            r0 = sid * (NP // 16) + t * CB
            pltpu.sync_copy(agg_sh.at[pl.ds(r0, CB)], rows_a)
            pltpu.sync_copy(rows_a, out_h.at[pl.ds(cid * NP + r0, CB)])

    return k(table, ea, src, dst, kf)


def _perm_gather(rank, x3o):
    mesh = plsc.VectorSubcoreMesh(core_axis_name="c", subcore_axis_name="s")
    rpt = KP // 32

    @functools.partial(
        pl.kernel,
        out_type=jax.ShapeDtypeStruct((KP, H), jnp.float32),
        mesh=mesh,
        compiler_params=pltpu.CompilerParams(needs_layout_passes=False),
        scratch_types=[
            pltpu.VMEM((NP,), jnp.int32),
            pltpu.VMEM((rpt // 2,), jnp.int32),
            pltpu.VMEM((rpt // 2,), jnp.int32),
            pltpu.VMEM((rpt, H), jnp.float32),
            pltpu.SemaphoreType.DMA,
        ],
    )
    def k(rank_h, x_h, out_h, rank_v, pa_v, pb_v, rows_v, sem):
        cid = lax.axis_index("c")
        sid = lax.axis_index("s")
        wid = cid * 16 + sid
        lo = wid * rpt
        hb = rpt // 2

        pltpu.sync_copy(rank_h, rank_v)
        for t in range(hb // 16):
            pa_v[pl.ds(t * 16, 16)] = jnp.zeros((16,), jnp.int32)
            pb_v[pl.ds(t * 16, 16)] = jnp.zeros((16,), jnp.int32)

        def scan(i, _):
            rv = rank_v[pl.ds(i * 16, 16)]
            iv = i * 16 + lax.broadcasted_iota(jnp.int32, (16,), 0)
            ma = jnp.logical_and(rv >= lo, rv < lo + hb)
            mb = jnp.logical_and(rv >= lo + hb, rv < lo + rpt)
            plsc.store_scatter(pa_v, [rv - lo], iv, mask=ma)
            plsc.store_scatter(pb_v, [rv - (lo + hb)], iv, mask=mb)
            return 0
        lax.fori_loop(0, NP // 16, scan, 0)

        g1 = pltpu.async_copy(x_h.at[pa_v], rows_v.at[pl.ds(0, hb)], sem)
        g2 = pltpu.async_copy(x_h.at[pb_v], rows_v.at[pl.ds(hb, hb)], sem)
        g1.wait()
        g2.wait()
        pltpu.sync_copy(rows_v, out_h.at[pl.ds(lo, rpt)])

    return k(rank, x3o)


def kernel(x, edge_index, edge_attr, Ws1, Wm1, We1, b1, pool_w,
           Ws2, Wm2, We2, b2, Ws3, Wm3, We3, b3):
    xp = jnp.pad(x, ((0, NP - N), (0, 0)))
    srcp = jnp.concatenate([edge_index[0],
                            jnp.zeros((EP - E,), jnp.int32)])
    dstp = jnp.concatenate([edge_index[1],
                            jnp.full((EP - E,), N, jnp.int32)])
    eap = jnp.pad(edge_attr, ((0, EP - E), (0, 0)))

    xm1 = _mm(xp, Wm1, _RB)
    ea1 = _mm(eap, We1, 2048)
    agg1 = _edge_pass_exact(xm1, ea1, srcp, dstp, b1)
    x1, score = _node_update_score(xp, Ws1, agg1, pool_w)

    xg, kf2, rank = _rank_gate(score, x1, pool_w)
    kf2 = kf2.reshape(NP)
    rank = rank.reshape(NP)

    xm2a = _mm(xg, Wm2[:, :H], _RB)
    xm2b = _mm(xg, Wm2[:, H:], _RB)
    ea2a = _mm_bias(eap, We2[:, :H], b2[:H], 2048)
    ea2b = _mm_bias(eap, We2[:, H:], b2[H:], 2048)
    agg2a = _edge_pass(xm2a, ea2a, srcp, dstp, kf2, H)
    agg2b = _edge_pass(xm2b, ea2b, srcp, dstp, kf2, H)
    x2 = _node_update2(xg, Ws2, agg2a[:NP], agg2a[NP:], agg2b[:NP], agg2b[NP:])

    xm3 = _mm(x2, Wm3, _RB)
    ea3 = _mm_bias(eap, We3, b3, 2048)
    agg3 = _edge_pass(xm3, ea3, srcp, dstp, kf2, H)
    x3o = _node_update3(x2, Ws3, agg3[:NP], agg3[NP:])

    x3p = _perm_gather(rank, x3o)
    return x3p[:KSEL]


# speedup vs baseline: 1.0456x; 1.0456x over previous
import functools

import jax
import jax.numpy as jnp
from jax import lax
from jax.experimental import pallas as pl
from jax.experimental.pallas import tpu as pltpu
from jax.experimental.pallas import tpu_sc as plsc

N = 10000
E = 320000
F = 128
DE = 16
H = 128
KSEL = N // 2

NP = 10240
EP = 323584
CB = 64
TE = EP // 32
NR = (NP // 16) // CB
DUMMY = N
KP = 5120


def _mm_body(a_ref, w_ref, o_ref):
    o_ref[...] = jnp.dot(a_ref[...], w_ref[...],
                         preferred_element_type=jnp.float32)


def _mm(a, w, rb):
    m, k = a.shape
    _, h = w.shape
    return pl.pallas_call(
        _mm_body,
        grid=(m // rb,),
        in_specs=[pl.BlockSpec((rb, k), lambda i: (i, 0)),
                  pl.BlockSpec((k, h), lambda i: (0, 0))],
        out_specs=pl.BlockSpec((rb, h), lambda i: (i, 0)),
        out_shape=jax.ShapeDtypeStruct((m, h), jnp.float32),
    )(a, w)


def _mm_bias_body(a_ref, w_ref, b_ref, o_ref):
    o_ref[...] = jnp.dot(a_ref[...], w_ref[...],
                         preferred_element_type=jnp.float32) + b_ref[...]


def _mm_bias(a, w, b, rb):
    m, k = a.shape
    _, h = w.shape
    return pl.pallas_call(
        _mm_bias_body,
        grid=(m // rb,),
        in_specs=[pl.BlockSpec((rb, k), lambda i: (i, 0)),
                  pl.BlockSpec((k, h), lambda i: (0, 0)),
                  pl.BlockSpec((1, h), lambda i: (0, 0))],
        out_specs=pl.BlockSpec((rb, h), lambda i: (i, 0)),
        out_shape=jax.ShapeDtypeStruct((m, h), jnp.float32),
    )(a, w, b.reshape(1, h))


_RB = 1024


def _nu_score_body(a_ref, w_ref, g0_ref, pw_ref, x_ref, s_ref):
    acc = jnp.dot(a_ref[...], w_ref[...], preferred_element_type=jnp.float32)
    x1 = jnp.maximum(acc + g0_ref[...], 0.0)
    x_ref[...] = x1
    s = jnp.dot(x1, pw_ref[...], preferred_element_type=jnp.float32)
    rows = pl.program_id(0) * _RB + lax.broadcasted_iota(jnp.int32, (_RB, 1), 0)
    s_ref[...] = jnp.where(rows < N, s, -1e30)


def _node_update_score(a, w, g0, pw):
    m, k = a.shape
    _, h = w.shape
    return pl.pallas_call(
        _nu_score_body,
        grid=(m // _RB,),
        in_specs=[pl.BlockSpec((_RB, k), lambda i: (i, 0)),
                  pl.BlockSpec((k, h), lambda i: (0, 0)),
                  pl.BlockSpec((_RB, h), lambda i: (i, 0)),
                  pl.BlockSpec((h, 1), lambda i: (0, 0))],
        out_specs=[pl.BlockSpec((_RB, h), lambda i: (i, 0)),
                   pl.BlockSpec((_RB, 1), lambda i: (i, 0))],
        out_shape=[jax.ShapeDtypeStruct((m, h), jnp.float32),
                   jax.ShapeDtypeStruct((m, 1), jnp.float32)],
    )(a, w, g0, pw.reshape(h, 1))


def _nu2_body(a_ref, w_ref, r0_ref, r1_ref, r2_ref, r3_ref, o_ref):
    acc = jnp.dot(a_ref[...], w_ref[...], preferred_element_type=jnp.float32)
    agg = jnp.concatenate([r0_ref[...] + r1_ref[...],
                           r2_ref[...] + r3_ref[...]], axis=1)
    o_ref[...] = jnp.maximum(acc + agg, 0.0)


def _node_update2(a, w, r0, r1, r2, r3):
    m, k = a.shape
    _, h = w.shape
    hh = h // 2
    return pl.pallas_call(
        _nu2_body,
        grid=(m // _RB,),
        in_specs=[pl.BlockSpec((_RB, k), lambda i: (i, 0)),
                  pl.BlockSpec((k, h), lambda i: (0, 0)),
                  pl.BlockSpec((_RB, hh), lambda i: (i, 0)),
                  pl.BlockSpec((_RB, hh), lambda i: (i, 0)),
                  pl.BlockSpec((_RB, hh), lambda i: (i, 0)),
                  pl.BlockSpec((_RB, hh), lambda i: (i, 0))],
        out_specs=pl.BlockSpec((_RB, h), lambda i: (i, 0)),
        out_shape=jax.ShapeDtypeStruct((m, h), jnp.float32),
    )(a, w, r0, r1, r2, r3)


def _nu3_body(a_ref, w_ref, r0_ref, r1_ref, o_ref):
    acc = jnp.dot(a_ref[...], w_ref[...], preferred_element_type=jnp.float32)
    o_ref[...] = jnp.maximum(acc + r0_ref[...] + r1_ref[...], 0.0)


def _node_update3(a, w, r0, r1):
    m, k = a.shape
    _, h = w.shape
    return pl.pallas_call(
        _nu3_body,
        grid=(m // _RB,),
        in_specs=[pl.BlockSpec((_RB, k), lambda i: (i, 0)),
                  pl.BlockSpec((k, h), lambda i: (0, 0)),
                  pl.BlockSpec((_RB, h), lambda i: (i, 0)),
                  pl.BlockSpec((_RB, h), lambda i: (i, 0))],
        out_specs=pl.BlockSpec((_RB, h), lambda i: (i, 0)),
        out_shape=jax.ShapeDtypeStruct((m, h), jnp.float32),
    )(a, w, r0, r1)


_JB = 1024


def _rank_body(scol_ref, srow_ref, x1_ref, pw_ref, xg_ref, kf_ref, rank_ref):
    si = scol_ref[...]
    ii = pl.program_id(0) * _RB + lax.broadcasted_iota(jnp.int32, (_RB, 1), 0)

    def body(jc, acc):
        sj = srow_ref[:, pl.ds(jc * _JB, _JB)]
        jj = jc * _JB + lax.broadcasted_iota(jnp.int32, (1, _JB), 1)
        gt = (sj > si).astype(jnp.int32)
        eq = jnp.logical_and(sj == si, jj < ii).astype(jnp.int32)
        return acc + jnp.sum(gt + eq, axis=1, keepdims=True)

    rank = lax.fori_loop(0, NP // _JB, body, jnp.zeros((_RB, 1), jnp.int32))
    rank_ref[...] = rank
    kf_ref[...] = (rank < KSEL).astype(jnp.int32)
    wn = jnp.sqrt(jnp.sum(pw_ref[...] ** 2))
    gate = jnp.tanh(si / (wn + 1e-16))
    xg_ref[...] = x1_ref[...] * gate


def _rank_gate(score, x1, pool_w):
    h = x1.shape[1]
    return pl.pallas_call(
        _rank_body,
        grid=(NP // _RB,),
        in_specs=[pl.BlockSpec((_RB, 1), lambda i: (i, 0)),
                  pl.BlockSpec((1, NP), lambda i: (0, 0)),
                  pl.BlockSpec((_RB, h), lambda i: (i, 0)),
                  pl.BlockSpec((1, h), lambda i: (0, 0))],
        out_specs=[pl.BlockSpec((_RB, h), lambda i: (i, 0)),
                   pl.BlockSpec((_RB, 1), lambda i: (i, 0)),
                   pl.BlockSpec((_RB, 1), lambda i: (i, 0))],
        out_shape=[jax.ShapeDtypeStruct((NP, h), jnp.float32),
                   jax.ShapeDtypeStruct((NP, 1), jnp.int32),
                   jax.ShapeDtypeStruct((NP, 1), jnp.int32)],
    )(score, score.reshape(1, NP), x1, pool_w.reshape(1, h))


def _mk_splits():
    bs = []
    for half in (0, 160000):
        for kk in range(1, 12):
            bs.append(half + kk * 10080)
        for mm in range(1, 5):
            bs.append(half + 110880 + mm * 9840)
    bs.append(160000)
    return tuple(sorted(bs))


_B_SPLITS = _mk_splits()
_NT = NP // 32
_ELCAP = 12288
_SCN = 512
_EB = 64


def _edge_pass_exact(table, ea, src, dst, b):
    mesh = plsc.VectorSubcoreMesh(core_axis_name="c", subcore_axis_name="s")

    @functools.partial(
        pl.kernel,
        out_type=jax.ShapeDtypeStruct((NP, H), jnp.float32),
        mesh=mesh,
        compiler_params=pltpu.CompilerParams(needs_layout_passes=False),
        scratch_types=[
            pltpu.VMEM((_NT, H), jnp.float32),
            pltpu.VMEM((_NT, H), jnp.float32),
            pltpu.VMEM((_ELCAP,), jnp.int32),
            pltpu.VMEM((_SCN,), jnp.int32),
            pltpu.VMEM((_SCN,), jnp.int32),
            pltpu.VMEM((_EB, H), jnp.float32),
            pltpu.VMEM((_EB, H), jnp.float32),
            pltpu.VMEM((_EB, H), jnp.float32),
            pltpu.VMEM((_EB, H), jnp.float32),
            pltpu.VMEM((_EB,), jnp.int32),
            pltpu.VMEM((_EB,), jnp.int32),
            pltpu.VMEM((_EB,), jnp.int32),
            pltpu.VMEM((_EB,), jnp.int32),
            pltpu.VMEM((_NT,), jnp.int32),
            pltpu.SMEM((4,), jnp.int32),
            pltpu.SMEM((4,), jnp.int32),
            pltpu.SMEM((4,), jnp.int32),
            pltpu.VMEM((H,), jnp.float32),
            pltpu.SemaphoreType.DMA,
            pltpu.SemaphoreType.DMA,
            pltpu.SemaphoreType.DMA,
        ],
    )
    def k(table_h, ea_h, src_h, dst_h, b_h, out_h,
          acc, part1, elist, dch_a, dch_b, rows_a, rows_b, ea_a, ea_b,
          sidx_a, sidx_b, didx_a, didx_b,
          hist, fl_node, fl_tgt, fl_seen, b_v, s1sem, rs_a, rs_b):
        cid = lax.axis_index("c")
        sid = lax.axis_index("s")
        wid = cid * 16 + sid
        lo = wid * _NT

        pltpu.sync_copy(b_h, b_v)

        def z2(i, _):
            for j in range(H // 16):
                acc[i, pl.ds(j * 16, 16)] = jnp.zeros((16,), jnp.float32)
                part1[i, pl.ds(j * 16, 16)] = jnp.zeros((16,), jnp.float32)
            return 0
        lax.fori_loop(0, _NT, z2, 0)
        for t in range(_NT // 16):
            hist[pl.ds(t * 16, 16)] = jnp.zeros((16,), jnp.int32)

        def zel(i, _):
            elist[pl.ds(i * 16, 16)] = jnp.zeros((16,), jnp.int32)
            return 0
        lax.fori_loop(0, _ELCAP // 16, zel, 0)
        for si in range(3):
            fl_node[si] = -1
            fl_tgt[si] = -1
            fl_seen[si] = 0

        def scan_issue(c, dch_x, sem_x):
            pltpu.async_copy(dst_h.at[pl.ds(c * _SCN, _SCN)], dch_x, sem_x)

        def scan_wait(c, dch_x, sem_x):
            pltpu.make_async_copy(
                dst_h.at[pl.ds(c * _SCN, _SCN)], dch_x, sem_x).wait()

        def scan_proc(c, dch_x, carry):
            def grp(g, carry2):
                wp2, cb2 = carry2
                dv = dch_x[pl.ds(g * 16, 16)]
                in_rng = jnp.logical_and(dv >= lo, dv < lo + _NT)
                in_rng = jnp.logical_and(in_rng, dv < N)
                plsc.addupdate_scatter(hist, [dv - lo],
                                       jnp.ones((16,), jnp.int32),
                                       mask=in_rng)
                below = plsc.all_reduce_population_count(dv < lo)[0]
                eids = (c * _SCN + g * 16
                        + lax.broadcasted_iota(jnp.int32, (16,), 0))
                plsc.store_compressed(elist.at[pl.ds(wp2, 16)], eids,
                                      mask=in_rng)
                nin = plsc.all_reduce_population_count(in_rng)[0]
                return (wp2 + nin, cb2 + below)
            return lax.fori_loop(0, _SCN // 16, grp, carry)

        scan_issue(0, dch_a, rs_a)

        def scan_pair(ii, carry):
            c0 = 2 * ii
            c1 = c0 + 1
            scan_wait(c0, dch_a, rs_a)
            scan_issue(c1, dch_b, rs_b)
            carry = scan_proc(c0, dch_a, carry)
            scan_wait(c1, dch_b, rs_b)
            scan_issue(c0 + 2, dch_a, rs_a)
            carry = scan_proc(c1, dch_b, carry)
            return carry
        nch = E // _SCN
        wp, cbelow = lax.fori_loop(0, (nch - 1) // 2, scan_pair,
                                   (jnp.int32(0), jnp.int32(0)))
        scan_wait(nch - 1, dch_a, rs_a)
        wp, cbelow = scan_proc(nch - 1, dch_a, (wp, cbelow))

        def ph2(t, carry):
            pos, kslot = carry
            h16 = hist[pl.ds(t * 16, 16)]
            inc = plsc.cumsum(h16)
            st = pos + (inc - h16)
            en = pos + inc
            csp = jnp.full((16,), -1, jnp.int32)
            for bt in _B_SPLITS:
                cond = jnp.logical_and(st < bt, bt < en)
                csp = jnp.where(cond, bt - st, csp)
            m = csp >= 0
            kcnt = plsc.all_reduce_population_count(m)[0]
            idx16 = t * 16 + lax.broadcasted_iota(jnp.int32, (16,), 0)
            nid = jnp.sum(jnp.where(m, idx16, 0))
            cc = jnp.sum(jnp.where(m, csp, 0))

            @pl.when(kcnt > 0)
            def _():
                fl_node[kslot] = nid
                fl_tgt[kslot] = cc
            return (pos + jnp.sum(h16), jnp.minimum(kslot + kcnt, 2))
        _, _ = lax.fori_loop(0, _NT // 16, ph2, (cbelow, jnp.int32(0)))
        f0n = fl_node[0]
        f1n = fl_node[1]
        f2n = fl_node[2]

        nblocks = (wp + _EB - 1) // _EB

        def issue_s1(base, sidx_x, didx_x, ea_x):
            il = elist.at[pl.ds(base, _EB)]
            d1 = pltpu.async_copy(src_h.at[il], sidx_x, s1sem)
            d2 = pltpu.async_copy(dst_h.at[il], didx_x, s1sem)
            d3 = pltpu.async_copy(ea_h.at[il], ea_x, s1sem)
            d1.wait()
            d2.wait()
            d3.wait()

        def issue_rows(sidx_x, rows_x, rs_x):
            pltpu.async_copy(table_h.at[sidx_x], rows_x, rs_x)

        def wait_rows(sidx_x, rows_x, rs_x):
            pltpu.make_async_copy(table_h.at[sidx_x], rows_x, rs_x).wait()

        def pedge_f(base, rows_x, ea_x, didx_x):
            def accum(ld, i):
                for jj in range(H // 16):
                    m = ((rows_x[i, pl.ds(jj * 16, 16)]
                          + ea_x[i, pl.ds(jj * 16, 16)])
                         + b_v[pl.ds(jj * 16, 16)])
                    acc[ld, pl.ds(jj * 16, 16)] = (
                        acc[ld, pl.ds(jj * 16, 16)]
                        + jnp.maximum(m, 0.0))

            def pedge_grp(g, _):
                gbase = base + g * 16
                dv16 = didx_x[pl.ds(g * 16, 16)] - lo
                hit16 = jnp.logical_or(
                    dv16 == f0n, jnp.logical_or(dv16 == f1n, dv16 == f2n))
                hit16i = hit16.astype(jnp.int32)
                nhit = plsc.all_reduce_population_count(hit16)[0]
                plain = jnp.logical_and(gbase + 16 <= wp, nhit == 0)

                @pl.when(plain)
                def _():
                    for j in range(16):
                        accum(dv16[j], g * 16 + j)

                @pl.when(jnp.logical_not(plain))
                def _():
                    for j in range(16):
                        i = g * 16 + j

                        @pl.when(gbase + j < wp)
                        def _(i=i, j=j):
                            ld = dv16[j]

                            @pl.when(hit16i[j] == 1)
                            def _():
                                slot = jnp.where(
                                    ld == f0n, 0,
                                    jnp.where(ld == f1n, 1, 2))
                                s = fl_seen[slot]

                                @pl.when(s == fl_tgt[slot])
                                def _():
                                    for jj in range(H // 16):
                                        part1[ld, pl.ds(jj * 16, 16)] = (
                                            part1[ld, pl.ds(jj * 16, 16)]
                                            + acc[ld, pl.ds(jj * 16, 16)])
                                        acc[ld, pl.ds(jj * 16, 16)] = (
                                            jnp.zeros((16,), jnp.float32))
                                fl_seen[slot] = s + 1
                            accum(ld, i)
                return 0
            lax.fori_loop(0, _EB // 16, pedge_grp, 0)

        issue_s1(0, sidx_a, didx_a, ea_a)
        issue_rows(sidx_a, rows_a, rs_a)

        def it(ii, _):
            b0 = 2 * ii
            b1 = b0 + 1
            wait_rows(sidx_a, rows_a, rs_a)

            @pl.when(b1 < nblocks)
            def _():
                issue_s1(b1 * _EB, sidx_b, didx_b, ea_b)
                issue_rows(sidx_b, rows_b, rs_b)
            pedge_f(b0 * _EB, rows_a, ea_a, didx_a)

            @pl.when(b1 < nblocks)
            def _():
                wait_rows(sidx_b, rows_b, rs_b)

                @pl.when(b1 + 1 < nblocks)
                def _():
                    issue_s1((b1 + 1) * _EB, sidx_a, didx_a, ea_a)
                    issue_rows(sidx_a, rows_a, rs_a)
                pedge_f(b1 * _EB, rows_b, ea_b, didx_b)
            return 0
        lax.fori_loop(0, (nblocks + 1) // 2, it, 0)

        def ep(t, _):
            def row(r, _):
                for j in range(H // 16):
                    rows_a[r, pl.ds(j * 16, 16)] = (
                        part1[t * _EB + r, pl.ds(j * 16, 16)]
                        + acc[t * _EB + r, pl.ds(j * 16, 16)])
                return 0
            lax.fori_loop(0, _EB, row, 0)
            pltpu.sync_copy(rows_a, out_h.at[pl.ds(lo + t * _EB, _EB)])
            return 0
        lax.fori_loop(0, _NT // _EB, ep, 0)

    return k(table, ea, src, dst, b)


def _edge_pass(table, ea, src, dst, kf, hh):
    mesh = plsc.VectorSubcoreMesh(core_axis_name="c", subcore_axis_name="s")

    @functools.partial(
        pl.kernel,
        out_type=jax.ShapeDtypeStruct((2 * NP, hh), jnp.float32),
        mesh=mesh,
        compiler_params=pltpu.CompilerParams(needs_layout_passes=False),
        scratch_types=[
            pltpu.VMEM((NP,), jnp.int32),
            pltpu.VMEM((CB,), jnp.int32),
            pltpu.VMEM((CB,), jnp.int32),
            pltpu.VMEM((CB,), jnp.int32),
            pltpu.VMEM((CB,), jnp.int32),
            pltpu.VMEM((CB,), jnp.int32),
            pltpu.VMEM((CB,), jnp.int32),
            pltpu.VMEM((CB, hh), jnp.float32),
            pltpu.VMEM((CB, hh), jnp.float32),
            pltpu.VMEM((CB, hh), jnp.float32),
            pltpu.VMEM((CB, hh), jnp.float32),
            pltpu.VMEM_SHARED((NP, hh), jnp.float32),
            pltpu.SemaphoreType.DMA,
            pltpu.SemaphoreType.DMA,
            pltpu.SemaphoreType.DMA,
            pltpu.SemaphoreType.DMA,
        ],
    )
    def k(table_h, ea_h, src_h, dst_h, kf_h, out_h,
          kf_v, src_a, src_b, dst_a, dst_b, idx_a, idx_b,
          rows_a, rows_b, ea_a, ea_b, agg_sh,
          s1_a, s1_b, rs_a, rs_b):
        cid = lax.axis_index("c")
        sid = lax.axis_index("s")
        wid = cid * 16 + sid
        nc = TE // CB

        pltpu.sync_copy(kf_h, kf_v)

        def zrow(i, _):
            for j in range(hh // 16):
                rows_a[i, pl.ds(j * 16, 16)] = jnp.zeros((16,), jnp.float32)
            return 0
        lax.fori_loop(0, CB, zrow, 0)
        for t in range(NR):
            pltpu.sync_copy(rows_a,
                            agg_sh.at[pl.ds(sid * (NP // 16) + t * CB, CB)])
        plsc.subcore_barrier()

        def issue_s1(ci, src_x, dst_x, ea_x, s1_x):
            ebase = wid * TE + ci * CB
            pltpu.async_copy(src_h.at[pl.ds(ebase, CB)], src_x, s1_x)
            pltpu.async_copy(dst_h.at[pl.ds(ebase, CB)], dst_x, s1_x)
            pltpu.async_copy(ea_h.at[pl.ds(ebase, CB)], ea_x, s1_x)

        def wait_s1(ci, src_x, dst_x, ea_x, s1_x):
            ebase = wid * TE + ci * CB
            pltpu.make_async_copy(src_h.at[pl.ds(ebase, CB)], src_x, s1_x).wait()
            pltpu.make_async_copy(dst_h.at[pl.ds(ebase, CB)], dst_x, s1_x).wait()
            pltpu.make_async_copy(ea_h.at[pl.ds(ebase, CB)], ea_x, s1_x).wait()

        issue_s1(0, src_a, dst_a, ea_a, s1_a)

        def it(ii, _):
            c0 = 2 * ii
            c1 = c0 + 1
            @pl.when(c0 == 0)
            def _():
                wait_s1(0, src_a, dst_a, ea_a, s1_a)
                pltpu.async_copy(table_h.at[src_a], rows_a, rs_a)
                issue_s1(1, src_b, dst_b, ea_b, s1_b)
                for j in range(CB // 16):
                    sv = src_a[pl.ds(j * 16, 16)]
                    dv = dst_a[pl.ds(j * 16, 16)]
                    ks = plsc.load_gather(kf_v, [sv])
                    kd = plsc.load_gather(kf_v, [dv])
                    ok = (ks + kd) == 2
                    idx_a[pl.ds(j * 16, 16)] = jnp.where(ok, dv, DUMMY + wid)
                pltpu.make_async_copy(table_h.at[src_a], rows_a, rs_a).wait()

                def mrow0(r, _):
                    for j in range(hh // 16):
                        v = (rows_a[r, pl.ds(j * 16, 16)]
                             + ea_a[r, pl.ds(j * 16, 16)])
                        ea_a[r, pl.ds(j * 16, 16)] = jnp.maximum(v, 0.0)
                    return 0
                lax.fori_loop(0, CB, mrow0, 0)
                pltpu.sync_copy(ea_a, agg_sh.at[idx_a], add=True)

            @pl.when(c0 > 0)
            def _():
                wait_s1(c0, src_a, dst_a, ea_a, s1_a)
                pltpu.async_copy(table_h.at[src_a], rows_a, rs_a)

                @pl.when(c1 < nc)
                def _():
                    issue_s1(c1, src_b, dst_b, ea_b, s1_b)
                for j in range(CB // 16):
                    sv = src_a[pl.ds(j * 16, 16)]
                    dv = dst_a[pl.ds(j * 16, 16)]
                    ks = plsc.load_gather(kf_v, [sv])
                    kd = plsc.load_gather(kf_v, [dv])
                    ok = (ks + kd) == 2
                    idx_a[pl.ds(j * 16, 16)] = jnp.where(ok, dv, DUMMY + wid)
                pltpu.make_async_copy(table_h.at[src_a], rows_a, rs_a).wait()

                def mrow1(r, _):
                    for j in range(hh // 16):
                        v = (rows_a[r, pl.ds(j * 16, 16)]
                             + ea_a[r, pl.ds(j * 16, 16)])
                        ea_a[r, pl.ds(j * 16, 16)] = jnp.maximum(v, 0.0)
                    return 0
                lax.fori_loop(0, CB, mrow1, 0)
                pltpu.sync_copy(ea_a, agg_sh.at[idx_a], add=True)

            @pl.when(c1 < nc)
            def _():
                wait_s1(c1, src_b, dst_b, ea_b, s1_b)
                pltpu.async_copy(table_h.at[src_b], rows_b, rs_b)

                @pl.when(c1 + 1 < nc)
                def _():
                    issue_s1(c1 + 1, src_a, dst_a, ea_a, s1_a)
                for j in range(CB // 16):
                    sv = src_b[pl.ds(j * 16, 16)]
                    dv = dst_b[pl.ds(j * 16, 16)]
                    ks = plsc.load_gather(kf_v, [sv])
                    kd = plsc.load_gather(kf_v, [dv])
                    ok = (ks + kd) == 2
                    idx_b[pl.ds(j * 16, 16)] = jnp.where(ok, dv, DUMMY + wid)
                pltpu.make_async_copy(table_h.at[src_b], rows_b, rs_b).wait()

                def mrow2(r, _):
                    for j in range(hh // 16):
                        v = (rows_b[r, pl.ds(j * 16, 16)]
                             + ea_b[r, pl.ds(j * 16, 16)])
                        ea_b[r, pl.ds(j * 16, 16)] = jnp.maximum(v, 0.0)
                    return 0
                lax.fori_loop(0, CB, mrow2, 0)
                pltpu.sync_copy(ea_b, agg_sh.at[idx_b], add=True)
            return 0
        lax.fori_loop(0, (nc + 1) // 2, it, 0)

        plsc.subcore_barrier()
        for t in range(NR):
            r0 = sid * (NP // 16) + t * CB
            pltpu.sync_copy(agg_sh.at[pl.ds(r0, CB)], rows_a)
            pltpu.sync_copy(rows_a, out_h.at[pl.ds(cid * NP + r0, CB)])

    return k(table, ea, src, dst, kf)


def _perm_gather(rank, x3o):
    mesh = plsc.VectorSubcoreMesh(core_axis_name="c", subcore_axis_name="s")
    rpt = KP // 32

    @functools.partial(
        pl.kernel,
        out_type=jax.ShapeDtypeStruct((KP, H), jnp.float32),
        mesh=mesh,
        compiler_params=pltpu.CompilerParams(needs_layout_passes=False),
        scratch_types=[
            pltpu.VMEM((NP,), jnp.int32),
            pltpu.VMEM((rpt // 2,), jnp.int32),
            pltpu.VMEM((rpt // 2,), jnp.int32),
            pltpu.VMEM((rpt, H), jnp.float32),
            pltpu.SemaphoreType.DMA,
        ],
    )
    def k(rank_h, x_h, out_h, rank_v, pa_v, pb_v, rows_v, sem):
        cid = lax.axis_index("c")
        sid = lax.axis_index("s")
        wid = cid * 16 + sid
        lo = wid * rpt
        hb = rpt // 2

        pltpu.sync_copy(rank_h, rank_v)
        for t in range(hb // 16):
            pa_v[pl.ds(t * 16, 16)] = jnp.zeros((16,), jnp.int32)
            pb_v[pl.ds(t * 16, 16)] = jnp.zeros((16,), jnp.int32)

        def scan(i, _):
            rv = rank_v[pl.ds(i * 16, 16)]
            iv = i * 16 + lax.broadcasted_iota(jnp.int32, (16,), 0)
            ma = jnp.logical_and(rv >= lo, rv < lo + hb)
            mb = jnp.logical_and(rv >= lo + hb, rv < lo + rpt)
            plsc.store_scatter(pa_v, [rv - lo], iv, mask=ma)
            plsc.store_scatter(pb_v, [rv - (lo + hb)], iv, mask=mb)
            return 0
        lax.fori_loop(0, NP // 16, scan, 0)

        g1 = pltpu.async_copy(x_h.at[pa_v], rows_v.at[pl.ds(0, hb)], sem)
        g2 = pltpu.async_copy(x_h.at[pb_v], rows_v.at[pl.ds(hb, hb)], sem)
        g1.wait()
        g2.wait()
        pltpu.sync_copy(rows_v, out_h.at[pl.ds(lo, rpt)])

    return k(rank, x3o)


def kernel(x, edge_index, edge_attr, Ws1, Wm1, We1, b1, pool_w,
           Ws2, Wm2, We2, b2, Ws3, Wm3, We3, b3):
    xp = jnp.pad(x, ((0, NP - N), (0, 0)))
    srcp = jnp.concatenate([edge_index[0],
                            jnp.zeros((EP - E,), jnp.int32)])
    dstp = jnp.concatenate([edge_index[1],
                            jnp.full((EP - E,), N, jnp.int32)])
    eap = jnp.pad(edge_attr, ((0, EP - E), (0, 0)))

    xm1 = _mm(xp, Wm1, _RB)
    ea1 = _mm(eap, We1, 2048)
    agg1 = _edge_pass_exact(xm1, ea1, srcp, dstp, b1)
    x1, score = _node_update_score(xp, Ws1, agg1, pool_w)

    xg, kf2, rank = _rank_gate(score, x1, pool_w)
    kf2 = kf2.reshape(NP)
    rank = rank.reshape(NP)

    xm2a = _mm(xg, Wm2[:, :H], _RB)
    xm2b = _mm(xg, Wm2[:, H:], _RB)
    ea2a = _mm_bias(eap, We2[:, :H], b2[:H], 2048)
    ea2b = _mm_bias(eap, We2[:, H:], b2[H:], 2048)
    agg2a = _edge_pass(xm2a, ea2a, srcp, dstp, kf2, H)
    agg2b = _edge_pass(xm2b, ea2b, srcp, dstp, kf2, H)
    x2 = _node_update2(xg, Ws2, agg2a[:NP], agg2a[NP:], agg2b[:NP], agg2b[NP:])

    xm3 = _mm(x2, Wm3, _RB)
    ea3 = _mm_bias(eap, We3, b3, 2048)
    agg3 = _edge_pass(xm3, ea3, srcp, dstp, kf2, H)
    x3o = _node_update3(x2, Ws3, agg3[:NP], agg3[NP:])

    x3p = _perm_gather(rank, x3o)
    return x3p[:KSEL]


# speedup vs baseline: 1.0565x; 1.0104x over previous
import functools

import jax
import jax.numpy as jnp
from jax import lax
from jax.experimental import pallas as pl
from jax.experimental.pallas import tpu as pltpu
from jax.experimental.pallas import tpu_sc as plsc

N = 10000
E = 320000
F = 128
DE = 16
H = 128
KSEL = N // 2

NP = 10240
EP = 323584
CB = 64
TE = EP // 32
NR = (NP // 16) // CB
DUMMY = N
KP = 5120


def _mm_body(a_ref, w_ref, o_ref):
    o_ref[...] = jnp.dot(a_ref[...], w_ref[...],
                         preferred_element_type=jnp.float32)


def _mm(a, w, rb):
    m, k = a.shape
    _, h = w.shape
    return pl.pallas_call(
        _mm_body,
        grid=(m // rb,),
        in_specs=[pl.BlockSpec((rb, k), lambda i: (i, 0)),
                  pl.BlockSpec((k, h), lambda i: (0, 0))],
        out_specs=pl.BlockSpec((rb, h), lambda i: (i, 0)),
        out_shape=jax.ShapeDtypeStruct((m, h), jnp.float32),
    )(a, w)


def _mm_bias_body(a_ref, w_ref, b_ref, o_ref):
    o_ref[...] = jnp.dot(a_ref[...], w_ref[...],
                         preferred_element_type=jnp.float32) + b_ref[...]


def _mm_bias(a, w, b, rb):
    m, k = a.shape
    _, h = w.shape
    return pl.pallas_call(
        _mm_bias_body,
        grid=(m // rb,),
        in_specs=[pl.BlockSpec((rb, k), lambda i: (i, 0)),
                  pl.BlockSpec((k, h), lambda i: (0, 0)),
                  pl.BlockSpec((1, h), lambda i: (0, 0))],
        out_specs=pl.BlockSpec((rb, h), lambda i: (i, 0)),
        out_shape=jax.ShapeDtypeStruct((m, h), jnp.float32),
    )(a, w, b.reshape(1, h))


_RB = 1024


def _nu_score_body(a_ref, w_ref, g0_ref, pw_ref, x_ref, s_ref):
    acc = jnp.dot(a_ref[...], w_ref[...], preferred_element_type=jnp.float32)
    x1 = jnp.maximum(acc + g0_ref[...], 0.0)
    x_ref[...] = x1
    s = jnp.dot(x1, pw_ref[...].reshape(-1, 1),
                preferred_element_type=jnp.float32)
    rows = pl.program_id(0) * _RB + lax.broadcasted_iota(jnp.int32, (_RB, 1), 0)
    s_ref[...] = jnp.where(rows < N, s, -1e30)


def _node_update_score(a, w, g0, pw):
    m, k = a.shape
    _, h = w.shape
    return pl.pallas_call(
        _nu_score_body,
        grid=(m // _RB,),
        in_specs=[pl.BlockSpec((_RB, k), lambda i: (i, 0)),
                  pl.BlockSpec((k, h), lambda i: (0, 0)),
                  pl.BlockSpec((_RB, h), lambda i: (i, 0)),
                  pl.BlockSpec((1, h), lambda i: (0, 0))],
        out_specs=[pl.BlockSpec((_RB, h), lambda i: (i, 0)),
                   pl.BlockSpec((_RB, 1), lambda i: (i, 0))],
        out_shape=[jax.ShapeDtypeStruct((m, h), jnp.float32),
                   jax.ShapeDtypeStruct((m, 1), jnp.float32)],
    )(a, w, g0, pw.reshape(1, h))


def _nu2_body(a_ref, w_ref, r0_ref, r1_ref, r2_ref, r3_ref, o_ref):
    acc = jnp.dot(a_ref[...], w_ref[...], preferred_element_type=jnp.float32)
    agg = jnp.concatenate([r0_ref[...] + r1_ref[...],
                           r2_ref[...] + r3_ref[...]], axis=1)
    o_ref[...] = jnp.maximum(acc + agg, 0.0)


def _node_update2(a, w, r0, r1, r2, r3):
    m, k = a.shape
    _, h = w.shape
    hh = h // 2
    return pl.pallas_call(
        _nu2_body,
        grid=(m // _RB,),
        in_specs=[pl.BlockSpec((_RB, k), lambda i: (i, 0)),
                  pl.BlockSpec((k, h), lambda i: (0, 0)),
                  pl.BlockSpec((_RB, hh), lambda i: (i, 0)),
                  pl.BlockSpec((_RB, hh), lambda i: (i, 0)),
                  pl.BlockSpec((_RB, hh), lambda i: (i, 0)),
                  pl.BlockSpec((_RB, hh), lambda i: (i, 0))],
        out_specs=pl.BlockSpec((_RB, h), lambda i: (i, 0)),
        out_shape=jax.ShapeDtypeStruct((m, h), jnp.float32),
    )(a, w, r0, r1, r2, r3)


def _nu3_body(a_ref, w_ref, r0_ref, r1_ref, o_ref):
    acc = jnp.dot(a_ref[...], w_ref[...], preferred_element_type=jnp.float32)
    o_ref[...] = jnp.maximum(acc + r0_ref[...] + r1_ref[...], 0.0)


def _node_update3(a, w, r0, r1):
    m, k = a.shape
    _, h = w.shape
    return pl.pallas_call(
        _nu3_body,
        grid=(m // _RB,),
        in_specs=[pl.BlockSpec((_RB, k), lambda i: (i, 0)),
                  pl.BlockSpec((k, h), lambda i: (0, 0)),
                  pl.BlockSpec((_RB, h), lambda i: (i, 0)),
                  pl.BlockSpec((_RB, h), lambda i: (i, 0))],
        out_specs=pl.BlockSpec((_RB, h), lambda i: (i, 0)),
        out_shape=jax.ShapeDtypeStruct((m, h), jnp.float32),
    )(a, w, r0, r1)


_JB = 1024


def _rank_body(scol_ref, srow_ref, x1_ref, pw_ref, xg_ref, kf_ref, rank_ref):
    si = scol_ref[...]
    ii = pl.program_id(0) * _RB + lax.broadcasted_iota(jnp.int32, (_RB, 1), 0)

    def body(jc, acc):
        sj = srow_ref[:, pl.ds(jc * _JB, _JB)]
        jj = jc * _JB + lax.broadcasted_iota(jnp.int32, (1, _JB), 1)
        gt = (sj > si).astype(jnp.int32)
        eq = jnp.logical_and(sj == si, jj < ii).astype(jnp.int32)
        return acc + jnp.sum(gt + eq, axis=1, keepdims=True)

    rank = lax.fori_loop(0, NP // _JB, body, jnp.zeros((_RB, 1), jnp.int32))
    rank_ref[...] = rank
    kf_ref[...] = (rank < KSEL).astype(jnp.int32)
    wn = jnp.sqrt(jnp.sum(pw_ref[...] ** 2))
    gate = jnp.tanh(si / (wn + 1e-16))
    xg_ref[...] = x1_ref[...] * gate


def _rank_gate(score, x1, pool_w):
    h = x1.shape[1]
    return pl.pallas_call(
        _rank_body,
        grid=(NP // _RB,),
        in_specs=[pl.BlockSpec((_RB, 1), lambda i: (i, 0)),
                  pl.BlockSpec((1, NP), lambda i: (0, 0)),
                  pl.BlockSpec((_RB, h), lambda i: (i, 0)),
                  pl.BlockSpec((1, h), lambda i: (0, 0))],
        out_specs=[pl.BlockSpec((_RB, h), lambda i: (i, 0)),
                   pl.BlockSpec((_RB, 1), lambda i: (i, 0)),
                   pl.BlockSpec((_RB, 1), lambda i: (i, 0))],
        out_shape=[jax.ShapeDtypeStruct((NP, h), jnp.float32),
                   jax.ShapeDtypeStruct((NP, 1), jnp.int32),
                   jax.ShapeDtypeStruct((NP, 1), jnp.int32)],
    )(score, score.reshape(1, NP), x1, pool_w.reshape(1, h))


def _mk_splits():
    bs = []
    for half in (0, 160000):
        for kk in range(1, 12):
            bs.append(half + kk * 10080)
        for mm in range(1, 5):
            bs.append(half + 110880 + mm * 9840)
    bs.append(160000)
    return tuple(sorted(bs))


_B_SPLITS = _mk_splits()
_NT = NP // 32
_ELCAP = 12288
_SCN = 512
_EB = 64


def _edge_pass_exact(table, ea, src, dst, b):
    mesh = plsc.VectorSubcoreMesh(core_axis_name="c", subcore_axis_name="s")

    @functools.partial(
        pl.kernel,
        out_type=jax.ShapeDtypeStruct((NP, H), jnp.float32),
        mesh=mesh,
        compiler_params=pltpu.CompilerParams(needs_layout_passes=False),
        scratch_types=[
            pltpu.VMEM((_NT, H), jnp.float32),
            pltpu.VMEM((_NT, H), jnp.float32),
            pltpu.VMEM((_ELCAP,), jnp.int32),
            pltpu.VMEM((_SCN,), jnp.int32),
            pltpu.VMEM((_SCN,), jnp.int32),
            pltpu.VMEM((_EB, H), jnp.float32),
            pltpu.VMEM((_EB, H), jnp.float32),
            pltpu.VMEM((_EB, H), jnp.float32),
            pltpu.VMEM((_EB, H), jnp.float32),
            pltpu.VMEM((_EB,), jnp.int32),
            pltpu.VMEM((_EB,), jnp.int32),
            pltpu.VMEM((_EB,), jnp.int32),
            pltpu.VMEM((_EB,), jnp.int32),
            pltpu.VMEM((_NT,), jnp.int32),
            pltpu.SMEM((4,), jnp.int32),
            pltpu.SMEM((4,), jnp.int32),
            pltpu.SMEM((4,), jnp.int32),
            pltpu.VMEM((H,), jnp.float32),
            pltpu.SemaphoreType.DMA,
            pltpu.SemaphoreType.DMA,
            pltpu.SemaphoreType.DMA,
        ],
    )
    def k(table_h, ea_h, src_h, dst_h, b_h, out_h,
          acc, part1, elist, dch_a, dch_b, rows_a, rows_b, ea_a, ea_b,
          sidx_a, sidx_b, didx_a, didx_b,
          hist, fl_node, fl_tgt, fl_seen, b_v, s1sem, rs_a, rs_b):
        cid = lax.axis_index("c")
        sid = lax.axis_index("s")
        wid = cid * 16 + sid
        lo = wid * _NT

        pltpu.sync_copy(b_h, b_v)

        def z2(i, _):
            for j in range(H // 16):
                acc[i, pl.ds(j * 16, 16)] = jnp.zeros((16,), jnp.float32)
                part1[i, pl.ds(j * 16, 16)] = jnp.zeros((16,), jnp.float32)
            return 0
        lax.fori_loop(0, _NT, z2, 0)
        for t in range(_NT // 16):
            hist[pl.ds(t * 16, 16)] = jnp.zeros((16,), jnp.int32)

        def zel(i, _):
            elist[pl.ds(i * 16, 16)] = jnp.zeros((16,), jnp.int32)
            return 0
        lax.fori_loop(0, _ELCAP // 16, zel, 0)
        for si in range(3):
            fl_node[si] = -1
            fl_tgt[si] = -1
            fl_seen[si] = 0

        def scan_issue(c, dch_x, sem_x):
            pltpu.async_copy(dst_h.at[pl.ds(c * _SCN, _SCN)], dch_x, sem_x)

        def scan_wait(c, dch_x, sem_x):
            pltpu.make_async_copy(
                dst_h.at[pl.ds(c * _SCN, _SCN)], dch_x, sem_x).wait()

        def scan_proc(c, dch_x, carry):
            def grp(g, carry2):
                wp2, cb2 = carry2
                dv = dch_x[pl.ds(g * 16, 16)]
                in_rng = jnp.logical_and(dv >= lo, dv < lo + _NT)
                in_rng = jnp.logical_and(in_rng, dv < N)
                plsc.addupdate_scatter(hist, [dv - lo],
                                       jnp.ones((16,), jnp.int32),
                                       mask=in_rng)
                below = plsc.all_reduce_population_count(dv < lo)[0]
                eids = (c * _SCN + g * 16
                        + lax.broadcasted_iota(jnp.int32, (16,), 0))
                plsc.store_compressed(elist.at[pl.ds(wp2, 16)], eids,
                                      mask=in_rng)
                nin = plsc.all_reduce_population_count(in_rng)[0]
                return (wp2 + nin, cb2 + below)
            return lax.fori_loop(0, _SCN // 16, grp, carry)

        scan_issue(0, dch_a, rs_a)

        def scan_pair(ii, carry):
            c0 = 2 * ii
            c1 = c0 + 1
            scan_wait(c0, dch_a, rs_a)
            scan_issue(c1, dch_b, rs_b)
            carry = scan_proc(c0, dch_a, carry)
            scan_wait(c1, dch_b, rs_b)
            scan_issue(c0 + 2, dch_a, rs_a)
            carry = scan_proc(c1, dch_b, carry)
            return carry
        nch = E // _SCN
        wp, cbelow = lax.fori_loop(0, (nch - 1) // 2, scan_pair,
                                   (jnp.int32(0), jnp.int32(0)))
        scan_wait(nch - 1, dch_a, rs_a)
        wp, cbelow = scan_proc(nch - 1, dch_a, (wp, cbelow))

        def ph2(t, carry):
            pos, kslot = carry
            h16 = hist[pl.ds(t * 16, 16)]
            inc = plsc.cumsum(h16)
            st = pos + (inc - h16)
            en = pos + inc
            csp = jnp.full((16,), -1, jnp.int32)
            for bt in _B_SPLITS:
                cond = jnp.logical_and(st < bt, bt < en)
                csp = jnp.where(cond, bt - st, csp)
            m = csp >= 0
            kcnt = plsc.all_reduce_population_count(m)[0]
            idx16 = t * 16 + lax.broadcasted_iota(jnp.int32, (16,), 0)
            nid = jnp.sum(jnp.where(m, idx16, 0))
            cc = jnp.sum(jnp.where(m, csp, 0))

            @pl.when(kcnt > 0)
            def _():
                fl_node[kslot] = nid
                fl_tgt[kslot] = cc
            return (pos + jnp.sum(h16), jnp.minimum(kslot + kcnt, 2))
        _, _ = lax.fori_loop(0, _NT // 16, ph2, (cbelow, jnp.int32(0)))
        f0n = fl_node[0]
        f1n = fl_node[1]
        f2n = fl_node[2]

        nblocks = (wp + _EB - 1) // _EB

        def issue_s1(base, sidx_x, didx_x, ea_x):
            il = elist.at[pl.ds(base, _EB)]
            d1 = pltpu.async_copy(src_h.at[il], sidx_x, s1sem)
            d2 = pltpu.async_copy(dst_h.at[il], didx_x, s1sem)
            d3 = pltpu.async_copy(ea_h.at[il], ea_x, s1sem)
            d1.wait()
            d2.wait()
            d3.wait()

        def issue_rows(sidx_x, rows_x, rs_x):
            pltpu.async_copy(table_h.at[sidx_x], rows_x, rs_x)

        def wait_rows(sidx_x, rows_x, rs_x):
            pltpu.make_async_copy(table_h.at[sidx_x], rows_x, rs_x).wait()

        def pedge_f(base, rows_x, ea_x, didx_x):
            def accum(ld, i):
                for jj in range(H // 16):
                    m = ((rows_x[i, pl.ds(jj * 16, 16)]
                          + ea_x[i, pl.ds(jj * 16, 16)])
                         + b_v[pl.ds(jj * 16, 16)])
                    acc[ld, pl.ds(jj * 16, 16)] = (
                        acc[ld, pl.ds(jj * 16, 16)]
                        + jnp.maximum(m, 0.0))

            def pedge_grp(g, _):
                gbase = base + g * 16
                dv16 = didx_x[pl.ds(g * 16, 16)] - lo
                hit16 = jnp.logical_or(
                    dv16 == f0n, jnp.logical_or(dv16 == f1n, dv16 == f2n))
                hit16i = hit16.astype(jnp.int32)
                nhit = plsc.all_reduce_population_count(hit16)[0]
                plain = jnp.logical_and(gbase + 16 <= wp, nhit == 0)

                @pl.when(plain)
                def _():
                    for j in range(16):
                        accum(dv16[j], g * 16 + j)

                @pl.when(jnp.logical_not(plain))
                def _():
                    for j in range(16):
                        i = g * 16 + j

                        @pl.when(gbase + j < wp)
                        def _(i=i, j=j):
                            ld = dv16[j]

                            @pl.when(hit16i[j] == 1)
                            def _():
                                slot = jnp.where(
                                    ld == f0n, 0,
                                    jnp.where(ld == f1n, 1, 2))
                                s = fl_seen[slot]

                                @pl.when(s == fl_tgt[slot])
                                def _():
                                    for jj in range(H // 16):
                                        part1[ld, pl.ds(jj * 16, 16)] = (
                                            part1[ld, pl.ds(jj * 16, 16)]
                                            + acc[ld, pl.ds(jj * 16, 16)])
                                        acc[ld, pl.ds(jj * 16, 16)] = (
                                            jnp.zeros((16,), jnp.float32))
                                fl_seen[slot] = s + 1
                            accum(ld, i)
                return 0
            lax.fori_loop(0, _EB // 16, pedge_grp, 0)

        issue_s1(0, sidx_a, didx_a, ea_a)
        issue_rows(sidx_a, rows_a, rs_a)

        def it(ii, _):
            b0 = 2 * ii
            b1 = b0 + 1
            wait_rows(sidx_a, rows_a, rs_a)

            @pl.when(b1 < nblocks)
            def _():
                issue_s1(b1 * _EB, sidx_b, didx_b, ea_b)
                issue_rows(sidx_b, rows_b, rs_b)
            pedge_f(b0 * _EB, rows_a, ea_a, didx_a)

            @pl.when(b1 < nblocks)
            def _():
                wait_rows(sidx_b, rows_b, rs_b)

                @pl.when(b1 + 1 < nblocks)
                def _():
                    issue_s1((b1 + 1) * _EB, sidx_a, didx_a, ea_a)
                    issue_rows(sidx_a, rows_a, rs_a)
                pedge_f(b1 * _EB, rows_b, ea_b, didx_b)
            return 0
        lax.fori_loop(0, (nblocks + 1) // 2, it, 0)

        def ep(t, _):
            def row(r, _):
                for j in range(H // 16):
                    rows_a[r, pl.ds(j * 16, 16)] = (
                        part1[t * _EB + r, pl.ds(j * 16, 16)]
                        + acc[t * _EB + r, pl.ds(j * 16, 16)])
                return 0
            lax.fori_loop(0, _EB, row, 0)
            pltpu.sync_copy(rows_a, out_h.at[pl.ds(lo + t * _EB, _EB)])
            return 0
        lax.fori_loop(0, _NT // _EB, ep, 0)

    return k(table, ea, src, dst, b)


def _edge_pass(table, ea, src, dst, kf, hh):
    mesh = plsc.VectorSubcoreMesh(core_axis_name="c", subcore_axis_name="s")

    @functools.partial(
        pl.kernel,
        out_type=jax.ShapeDtypeStruct((2 * NP, hh), jnp.float32),
        mesh=mesh,
        compiler_params=pltpu.CompilerParams(needs_layout_passes=False),
        scratch_types=[
            pltpu.VMEM((NP,), jnp.int32),
            pltpu.VMEM((CB,), jnp.int32),
            pltpu.VMEM((CB,), jnp.int32),
            pltpu.VMEM((CB,), jnp.int32),
            pltpu.VMEM((CB,), jnp.int32),
            pltpu.VMEM((CB,), jnp.int32),
            pltpu.VMEM((CB,), jnp.int32),
            pltpu.VMEM((CB, hh), jnp.float32),
            pltpu.VMEM((CB, hh), jnp.float32),
            pltpu.VMEM((CB, hh), jnp.float32),
            pltpu.VMEM((CB, hh), jnp.float32),
            pltpu.VMEM_SHARED((NP, hh), jnp.float32),
            pltpu.SemaphoreType.DMA,
            pltpu.SemaphoreType.DMA,
            pltpu.SemaphoreType.DMA,
            pltpu.SemaphoreType.DMA,
        ],
    )
    def k(table_h, ea_h, src_h, dst_h, kf_h, out_h,
          kf_v, src_a, src_b, dst_a, dst_b, idx_a, idx_b,
          rows_a, rows_b, ea_a, ea_b, agg_sh,
          s1_a, s1_b, rs_a, rs_b):
        cid = lax.axis_index("c")
        sid = lax.axis_index("s")
        wid = cid * 16 + sid
        nc = TE // CB

        pltpu.sync_copy(kf_h, kf_v)

        def zrow(i, _):
            for j in range(hh // 16):
                rows_a[i, pl.ds(j * 16, 16)] = jnp.zeros((16,), jnp.float32)
            return 0
        lax.fori_loop(0, CB, zrow, 0)
        for t in range(NR):
            pltpu.sync_copy(rows_a,
                            agg_sh.at[pl.ds(sid * (NP // 16) + t * CB, CB)])
        plsc.subcore_barrier()

        def issue_s1(ci, src_x, dst_x, ea_x, s1_x):
            ebase = wid * TE + ci * CB
            pltpu.async_copy(src_h.at[pl.ds(ebase, CB)], src_x, s1_x)
            pltpu.async_copy(dst_h.at[pl.ds(ebase, CB)], dst_x, s1_x)
            pltpu.async_copy(ea_h.at[pl.ds(ebase, CB)], ea_x, s1_x)

        def wait_s1(ci, src_x, dst_x, ea_x, s1_x):
            ebase = wid * TE + ci * CB
            pltpu.make_async_copy(src_h.at[pl.ds(ebase, CB)], src_x, s1_x).wait()
            pltpu.make_async_copy(dst_h.at[pl.ds(ebase, CB)], dst_x, s1_x).wait()
            pltpu.make_async_copy(ea_h.at[pl.ds(ebase, CB)], ea_x, s1_x).wait()

        issue_s1(0, src_a, dst_a, ea_a, s1_a)

        def it(ii, _):
            c0 = 2 * ii
            c1 = c0 + 1
            @pl.when(c0 == 0)
            def _():
                wait_s1(0, src_a, dst_a, ea_a, s1_a)
                pltpu.async_copy(table_h.at[src_a], rows_a, rs_a)
                issue_s1(1, src_b, dst_b, ea_b, s1_b)
                for j in range(CB // 16):
                    sv = src_a[pl.ds(j * 16, 16)]
                    dv = dst_a[pl.ds(j * 16, 16)]
                    ks = plsc.load_gather(kf_v, [sv])
                    kd = plsc.load_gather(kf_v, [dv])
                    ok = (ks + kd) == 2
                    idx_a[pl.ds(j * 16, 16)] = jnp.where(ok, dv, DUMMY + wid)
                pltpu.make_async_copy(table_h.at[src_a], rows_a, rs_a).wait()

                def mrow0(r, _):
                    for j in range(hh // 16):
                        v = (rows_a[r, pl.ds(j * 16, 16)]
                             + ea_a[r, pl.ds(j * 16, 16)])
                        ea_a[r, pl.ds(j * 16, 16)] = jnp.maximum(v, 0.0)
                    return 0
                lax.fori_loop(0, CB, mrow0, 0)
                pltpu.sync_copy(ea_a, agg_sh.at[idx_a], add=True)

            @pl.when(c0 > 0)
            def _():
                wait_s1(c0, src_a, dst_a, ea_a, s1_a)
                pltpu.async_copy(table_h.at[src_a], rows_a, rs_a)

                @pl.when(c1 < nc)
                def _():
                    issue_s1(c1, src_b, dst_b, ea_b, s1_b)
                for j in range(CB // 16):
                    sv = src_a[pl.ds(j * 16, 16)]
                    dv = dst_a[pl.ds(j * 16, 16)]
                    ks = plsc.load_gather(kf_v, [sv])
                    kd = plsc.load_gather(kf_v, [dv])
                    ok = (ks + kd) == 2
                    idx_a[pl.ds(j * 16, 16)] = jnp.where(ok, dv, DUMMY + wid)
                pltpu.make_async_copy(table_h.at[src_a], rows_a, rs_a).wait()

                def mrow1(r, _):
                    for j in range(hh // 16):
                        v = (rows_a[r, pl.ds(j * 16, 16)]
                             + ea_a[r, pl.ds(j * 16, 16)])
                        ea_a[r, pl.ds(j * 16, 16)] = jnp.maximum(v, 0.0)
                    return 0
                lax.fori_loop(0, CB, mrow1, 0)
                pltpu.sync_copy(ea_a, agg_sh.at[idx_a], add=True)

            @pl.when(c1 < nc)
            def _():
                wait_s1(c1, src_b, dst_b, ea_b, s1_b)
                pltpu.async_copy(table_h.at[src_b], rows_b, rs_b)

                @pl.when(c1 + 1 < nc)
                def _():
                    issue_s1(c1 + 1, src_a, dst_a, ea_a, s1_a)
                for j in range(CB // 16):
                    sv = src_b[pl.ds(j * 16, 16)]
                    dv = dst_b[pl.ds(j * 16, 16)]
                    ks = plsc.load_gather(kf_v, [sv])
                    kd = plsc.load_gather(kf_v, [dv])
                    ok = (ks + kd) == 2
                    idx_b[pl.ds(j * 16, 16)] = jnp.where(ok, dv, DUMMY + wid)
                pltpu.make_async_copy(table_h.at[src_b], rows_b, rs_b).wait()

                def mrow2(r, _):
                    for j in range(hh // 16):
                        v = (rows_b[r, pl.ds(j * 16, 16)]
                             + ea_b[r, pl.ds(j * 16, 16)])
                        ea_b[r, pl.ds(j * 16, 16)] = jnp.maximum(v, 0.0)
                    return 0
                lax.fori_loop(0, CB, mrow2, 0)
                pltpu.sync_copy(ea_b, agg_sh.at[idx_b], add=True)
            return 0
        lax.fori_loop(0, (nc + 1) // 2, it, 0)

        plsc.subcore_barrier()
        for t in range(NR):
            r0 = sid * (NP // 16) + t * CB
            pltpu.sync_copy(agg_sh.at[pl.ds(r0, CB)], rows_a)
            pltpu.sync_copy(rows_a, out_h.at[pl.ds(cid * NP + r0, CB)])

    return k(table, ea, src, dst, kf)


def _perm_gather(rank, x3o):
    mesh = plsc.VectorSubcoreMesh(core_axis_name="c", subcore_axis_name="s")
    rpt = KP // 32

    @functools.partial(
        pl.kernel,
        out_type=jax.ShapeDtypeStruct((KP, H), jnp.float32),
        mesh=mesh,
        compiler_params=pltpu.CompilerParams(needs_layout_passes=False),
        scratch_types=[
            pltpu.VMEM((NP,), jnp.int32),
            pltpu.VMEM((rpt // 2,), jnp.int32),
            pltpu.VMEM((rpt // 2,), jnp.int32),
            pltpu.VMEM((rpt, H), jnp.float32),
            pltpu.SemaphoreType.DMA,
        ],
    )
    def k(rank_h, x_h, out_h, rank_v, pa_v, pb_v, rows_v, sem):
        cid = lax.axis_index("c")
        sid = lax.axis_index("s")
        wid = cid * 16 + sid
        lo = wid * rpt
        hb = rpt // 2

        pltpu.sync_copy(rank_h, rank_v)
        for t in range(hb // 16):
            pa_v[pl.ds(t * 16, 16)] = jnp.zeros((16,), jnp.int32)
            pb_v[pl.ds(t * 16, 16)] = jnp.zeros((16,), jnp.int32)

        def scan(i, _):
            rv = rank_v[pl.ds(i * 16, 16)]
            iv = i * 16 + lax.broadcasted_iota(jnp.int32, (16,), 0)
            ma = jnp.logical_and(rv >= lo, rv < lo + hb)
            mb = jnp.logical_and(rv >= lo + hb, rv < lo + rpt)
            plsc.store_scatter(pa_v, [rv - lo], iv, mask=ma)
            plsc.store_scatter(pb_v, [rv - (lo + hb)], iv, mask=mb)
            return 0
        lax.fori_loop(0, NP // 16, scan, 0)

        g1 = pltpu.async_copy(x_h.at[pa_v], rows_v.at[pl.ds(0, hb)], sem)
        g2 = pltpu.async_copy(x_h.at[pb_v], rows_v.at[pl.ds(hb, hb)], sem)
        g1.wait()
        g2.wait()
        pltpu.sync_copy(rows_v, out_h.at[pl.ds(lo, rpt)])

    return k(rank, x3o)


def kernel(x, edge_index, edge_attr, Ws1, Wm1, We1, b1, pool_w,
           Ws2, Wm2, We2, b2, Ws3, Wm3, We3, b3):
    xp = jnp.pad(x, ((0, NP - N), (0, 0)))
    srcp = jnp.concatenate([edge_index[0],
                            jnp.zeros((EP - E,), jnp.int32)])
    dstp = jnp.concatenate([edge_index[1],
                            jnp.full((EP - E,), N, jnp.int32)])
    eap = jnp.pad(edge_attr, ((0, EP - E), (0, 0)))

    xm1 = _mm(xp, Wm1, _RB)
    ea1 = _mm(eap, We1, 2048)
    agg1 = _edge_pass_exact(xm1, ea1, srcp, dstp, b1)
    x1, score = _node_update_score(xp, Ws1, agg1, pool_w)

    xg, kf2, rank = _rank_gate(score, x1, pool_w)
    kf2 = kf2.reshape(NP)
    rank = rank.reshape(NP)

    xm2a = _mm(xg, Wm2[:, :H], _RB)
    xm2b = _mm(xg, Wm2[:, H:], _RB)
    ea2a = _mm_bias(eap, We2[:, :H], b2[:H], 2048)
    ea2b = _mm_bias(eap, We2[:, H:], b2[H:], 2048)
    agg2a = _edge_pass(xm2a, ea2a, srcp, dstp, kf2, H)
    agg2b = _edge_pass(xm2b, ea2b, srcp, dstp, kf2, H)
    x2 = _node_update2(xg, Ws2, agg2a[:NP], agg2a[NP:], agg2b[:NP], agg2b[NP:])

    xm3 = _mm(x2, Wm3, _RB)
    ea3 = _mm_bias(eap, We3, b3, 2048)
    agg3 = _edge_pass(xm3, ea3, srcp, dstp, kf2, H)
    x3o = _node_update3(x2, Ws3, agg3[:NP], agg3[NP:])

    x3p = _perm_gather(rank, x3o)
    return x3p[:KSEL]
